# baseline (speedup 1.0000x reference)
import functools

import jax
import jax.numpy as jnp
from jax import lax
from jax.experimental import pallas as pl
from jax.experimental.pallas import tpu as pltpu
from jax.experimental.pallas import tpu_sc as plsc

N = 10000
E = 320000
IN_DIM = 128
D = 128

NC = 2
NS = 16
CHUNK = 125
PHASES = 5
CPP = 16
CPW = PHASES * CPP
NBUF = 2
NP = 10240
ZR = NP // NS

_MM_BLK = 2000


def _mm_body(h_ref, w_ref, b_ref, n_ref, o_ref):
    acc = jnp.dot(h_ref[...], w_ref[...], preferred_element_type=jnp.float32)
    o_ref[...] = (acc + b_ref[...]) * n_ref[...]


def _matmul(h, wcat, bcat, norm):
    return pl.pallas_call(
        _mm_body,
        grid=(N // _MM_BLK,),
        in_specs=[
            pl.BlockSpec((_MM_BLK, IN_DIM), lambda i: (i, 0)),
            pl.BlockSpec((IN_DIM, D), lambda i: (0, 0)),
            pl.BlockSpec((1, D), lambda i: (0, 0)),
            pl.BlockSpec((_MM_BLK, 1), lambda i: (i, 0)),
        ],
        out_specs=pl.BlockSpec((_MM_BLK, D), lambda i: (i, 0)),
        out_shape=jax.ShapeDtypeStruct((N, D), jnp.float32),
    )(h, wcat, bcat, norm)


def _fin_body(p_ref, n_ref, o_ref):
    o_ref[...] = (p_ref[0] + p_ref[1]) * n_ref[...]


def _combine(parts, norm):
    return pl.pallas_call(
        _fin_body,
        grid=(N // _MM_BLK,),
        in_specs=[
            pl.BlockSpec((2, _MM_BLK, D), lambda i: (0, i, 0)),
            pl.BlockSpec((_MM_BLK, 1), lambda i: (i, 0)),
        ],
        out_specs=pl.BlockSpec((_MM_BLK, D), lambda i: (i, 0)),
        out_shape=jax.ShapeDtypeStruct((N, D), jnp.float32),
    )(parts, norm)


_SC_MESH = plsc.VectorSubcoreMesh(
    core_axis_name="c", subcore_axis_name="s", num_cores=NC, num_subcores=NS)


@functools.partial(
    pl.kernel,
    out_type=jax.ShapeDtypeStruct((NC, NP, D), jnp.float32),
    mesh=_SC_MESH,
    scratch_types=[
        pltpu.VMEM((CPP, CHUNK), jnp.int32),
        pltpu.VMEM((CPP, CHUNK), jnp.int32),
        [pltpu.VMEM((CHUNK, D), jnp.float32) for _ in range(NBUF)],
        pltpu.VMEM_SHARED((NP, D), jnp.float32),
        [pltpu.SemaphoreType.DMA for _ in range(NBUF)],
    ],
)
def _sc_scatter(hi_hbm, src_hbm, dst_hbm, zero_hbm, out_hbm,
                src_v, dst_v, rows_v, acc, sems):
    cid = lax.axis_index("c")
    sid = lax.axis_index("s")
    pltpu.sync_copy(zero_hbm, acc.at[pl.ds(sid * ZR, ZR)])
    plsc.subcore_barrier()

    for p in range(PHASES):
        pltpu.sync_copy(src_hbm.at[cid, sid, p], src_v)
        pltpu.sync_copy(dst_hbm.at[cid, sid, p], dst_v)
        for b in range(NBUF):
            pltpu.async_copy(hi_hbm.at[src_v.at[b]], rows_v[b], sems[b])

        def body(g, carry):
            for b in range(NBUF):
                j = g * NBUF + b
                pltpu.make_async_copy(
                    hi_hbm.at[src_v.at[j]], rows_v[b], sems[b]).wait()
                pltpu.sync_copy(rows_v[b], acc.at[dst_v.at[j]], add=True)

                @pl.when(j + NBUF < CPP)
                def _():
                    pltpu.async_copy(
                        hi_hbm.at[src_v.at[j + NBUF]], rows_v[b], sems[b])
            return carry

        lax.fori_loop(0, CPP // NBUF, body, 0)
    plsc.subcore_barrier()
    pltpu.sync_copy(acc.at[pl.ds(sid * ZR, ZR)],
                    out_hbm.at[cid, pl.ds(sid * ZR, ZR)])


def kernel(t, h, edge_index, norm, W, b):
    del t
    wcat = W.transpose(1, 0, 2).reshape(IN_DIM, D)
    bcat = b.reshape(1, D)
    hi = _matmul(h, wcat, bcat, norm)

    src = edge_index[0].reshape(NC, NS, PHASES, CPP, CHUNK)
    dst = edge_index[1].reshape(NC, NS, PHASES, CPP, CHUNK)
    zero = jnp.zeros((ZR, D), jnp.float32)

    parts = _sc_scatter(hi, src, dst, zero)
    return _combine(parts, norm)


# speedup vs baseline: 1.0805x; 1.0805x over previous
import functools

import jax
import jax.numpy as jnp
from jax import lax
from jax.experimental import pallas as pl
from jax.experimental.pallas import tpu as pltpu
from jax.experimental.pallas import tpu_sc as plsc

N = 10000
E = 320000
IN_DIM = 128
D = 128

NC = 2
NS = 16
CHUNK = 125
PHASES = 5
CPP = 16
CPW = PHASES * CPP
NBUF = 2
NP = 10240
ZR = NP // NS

_MM_BLK = 5000


def _mm_body(h_ref, w_ref, b_ref, n_ref, o_ref):
    acc = jnp.dot(h_ref[...], w_ref[...], preferred_element_type=jnp.float32)
    o_ref[...] = (acc + b_ref[...]) * n_ref[...]


def _matmul(h, wcat, bcat, norm):
    return pl.pallas_call(
        _mm_body,
        grid=(N // _MM_BLK,),
        in_specs=[
            pl.BlockSpec((_MM_BLK, IN_DIM), lambda i: (i, 0)),
            pl.BlockSpec((IN_DIM, D), lambda i: (0, 0)),
            pl.BlockSpec((1, D), lambda i: (0, 0)),
            pl.BlockSpec((_MM_BLK, 1), lambda i: (i, 0)),
        ],
        out_specs=pl.BlockSpec((_MM_BLK, D), lambda i: (i, 0)),
        out_shape=jax.ShapeDtypeStruct((N, D), jnp.float32),
    )(h, wcat, bcat, norm)


def _fin_body(p_ref, n_ref, o_ref):
    o_ref[...] = (p_ref[0] + p_ref[1]) * n_ref[...]


def _combine(parts, norm):
    return pl.pallas_call(
        _fin_body,
        grid=(N // _MM_BLK,),
        in_specs=[
            pl.BlockSpec((2, _MM_BLK, D), lambda i: (0, i, 0)),
            pl.BlockSpec((_MM_BLK, 1), lambda i: (i, 0)),
        ],
        out_specs=pl.BlockSpec((_MM_BLK, D), lambda i: (i, 0)),
        out_shape=jax.ShapeDtypeStruct((N, D), jnp.float32),
    )(parts, norm)


_SC_MESH = plsc.VectorSubcoreMesh(
    core_axis_name="c", subcore_axis_name="s", num_cores=NC, num_subcores=NS)


@functools.partial(
    pl.kernel,
    out_type=jax.ShapeDtypeStruct((NC, NP, D), jnp.float32),
    mesh=_SC_MESH,
    scratch_types=[
        pltpu.VMEM((CPP, CHUNK), jnp.int32),
        pltpu.VMEM((CPP, CHUNK), jnp.int32),
        [pltpu.VMEM((CHUNK, D), jnp.float32) for _ in range(NBUF)],
        pltpu.VMEM_SHARED((NP, D), jnp.float32),
        [pltpu.SemaphoreType.DMA for _ in range(NBUF)],
    ],
)
def _sc_scatter(hi_hbm, edges_hbm, zero_hbm, out_hbm,
                src_v, dst_v, rows_v, acc, sems):
    cid = lax.axis_index("c")
    sid = lax.axis_index("s")
    pltpu.sync_copy(zero_hbm, acc.at[pl.ds(sid * ZR, ZR)])
    plsc.subcore_barrier()

    for p in range(PHASES):
        pltpu.sync_copy(edges_hbm.at[0, cid, sid, p], src_v)
        pltpu.sync_copy(edges_hbm.at[1, cid, sid, p], dst_v)
        for b in range(NBUF):
            pltpu.async_copy(hi_hbm.at[src_v.at[b]], rows_v[b], sems[b])

        def body(g, carry):
            for b in range(NBUF):
                j = g * NBUF + b
                pltpu.make_async_copy(
                    hi_hbm.at[src_v.at[j]], rows_v[b], sems[b]).wait()
                pltpu.sync_copy(rows_v[b], acc.at[dst_v.at[j]], add=True)

                @pl.when(j + NBUF < CPP)
                def _():
                    pltpu.async_copy(
                        hi_hbm.at[src_v.at[j + NBUF]], rows_v[b], sems[b])
            return carry

        lax.fori_loop(0, CPP // NBUF, body, 0)
    plsc.subcore_barrier()
    pltpu.sync_copy(acc.at[pl.ds(sid * ZR, ZR)],
                    out_hbm.at[cid, pl.ds(sid * ZR, ZR)])


def kernel(t, h, edge_index, norm, W, b):
    del t
    wcat = W.transpose(1, 0, 2).reshape(IN_DIM, D)
    bcat = b.reshape(1, D)
    hi = _matmul(h, wcat, bcat, norm)

    edges = edge_index.reshape(2, NC, NS, PHASES, CPP, CHUNK)
    zero = jnp.zeros((ZR, D), jnp.float32)

    parts = _sc_scatter(hi, edges, zero)
    return _combine(parts, norm)


# speedup vs baseline: 1.1231x; 1.0394x over previous
import functools

import jax
import jax.numpy as jnp
from jax import lax
from jax.experimental import pallas as pl
from jax.experimental.pallas import tpu as pltpu
from jax.experimental.pallas import tpu_sc as plsc

N = 10000
E = 320000
IN_DIM = 128
D = 128

NC = 2
NS = 16
CHUNK = 125
PHASES = 4
CPP = 20
CPW = PHASES * CPP
NBUF = 2
NP = 10240
ZR = NP // NS

_MM_BLK = 5000


def _mm_body(h_ref, w_ref, b_ref, n_ref, o_ref):
    acc = jnp.dot(h_ref[...], w_ref[...], preferred_element_type=jnp.float32)
    o_ref[...] = (acc + b_ref[...]) * n_ref[...]


def _matmul(h, wcat, bcat, norm):
    return pl.pallas_call(
        _mm_body,
        grid=(N // _MM_BLK,),
        in_specs=[
            pl.BlockSpec((_MM_BLK, IN_DIM), lambda i: (i, 0)),
            pl.BlockSpec((IN_DIM, D), lambda i: (0, 0)),
            pl.BlockSpec((1, D), lambda i: (0, 0)),
            pl.BlockSpec((_MM_BLK, 1), lambda i: (i, 0)),
        ],
        out_specs=pl.BlockSpec((_MM_BLK, D), lambda i: (i, 0)),
        out_shape=jax.ShapeDtypeStruct((N, D), jnp.float32),
    )(h, wcat, bcat, norm)


def _fin_body(p_ref, n_ref, o_ref):
    o_ref[...] = (p_ref[0] + p_ref[1]) * n_ref[...]


def _combine(parts, norm):
    return pl.pallas_call(
        _fin_body,
        grid=(N // _MM_BLK,),
        in_specs=[
            pl.BlockSpec((2, _MM_BLK, D), lambda i: (0, i, 0)),
            pl.BlockSpec((_MM_BLK, 1), lambda i: (i, 0)),
        ],
        out_specs=pl.BlockSpec((_MM_BLK, D), lambda i: (i, 0)),
        out_shape=jax.ShapeDtypeStruct((N, D), jnp.float32),
    )(parts, norm)


_SC_MESH = plsc.VectorSubcoreMesh(
    core_axis_name="c", subcore_axis_name="s", num_cores=NC, num_subcores=NS)


@functools.partial(
    pl.kernel,
    out_type=jax.ShapeDtypeStruct((NC, NP, D), jnp.float32),
    mesh=_SC_MESH,
    scratch_types=[
        pltpu.VMEM((CPP, CHUNK), jnp.int32),
        pltpu.VMEM((CPP, CHUNK), jnp.int32),
        [pltpu.VMEM((CHUNK, D), jnp.float32) for _ in range(NBUF)],
        pltpu.VMEM_SHARED((NP, D), jnp.float32),
        [pltpu.SemaphoreType.DMA for _ in range(NBUF)],
    ],
)
def _sc_scatter(hi_hbm, edges_hbm, zero_hbm, out_hbm,
                src_v, dst_v, rows_v, acc, sems):
    cid = lax.axis_index("c")
    sid = lax.axis_index("s")
    pltpu.sync_copy(zero_hbm, acc.at[pl.ds(sid * ZR, ZR)])
    plsc.subcore_barrier()

    for p in range(PHASES):
        pltpu.sync_copy(edges_hbm.at[0, cid, sid, p], src_v)
        pltpu.sync_copy(edges_hbm.at[1, cid, sid, p], dst_v)
        for b in range(NBUF):
            pltpu.async_copy(hi_hbm.at[src_v.at[b]], rows_v[b], sems[b])

        def body(g, carry):
            for b in range(NBUF):
                j = g * NBUF + b
                pltpu.make_async_copy(
                    hi_hbm.at[src_v.at[j]], rows_v[b], sems[b]).wait()
                pltpu.sync_copy(rows_v[b], acc.at[dst_v.at[j]], add=True)

                @pl.when(j + NBUF < CPP)
                def _():
                    pltpu.async_copy(
                        hi_hbm.at[src_v.at[j + NBUF]], rows_v[b], sems[b])
            return carry

        lax.fori_loop(0, CPP // NBUF, body, 0)
    plsc.subcore_barrier()
    pltpu.sync_copy(acc.at[pl.ds(sid * ZR, ZR)],
                    out_hbm.at[cid, pl.ds(sid * ZR, ZR)])


def kernel(t, h, edge_index, norm, W, b):
    del t
    wcat = W.transpose(1, 0, 2).reshape(IN_DIM, D)
    bcat = b.reshape(1, D)
    hi = _matmul(h, wcat, bcat, norm)

    edges = edge_index.reshape(2, NC, NS, PHASES, CPP, CHUNK)
    zero = jnp.zeros((ZR, D), jnp.float32)

    parts = _sc_scatter(hi, edges, zero)
    return _combine(parts, norm)
